# Initial kernel scaffold; baseline (speedup 1.0000x reference)
import jax
import jax.numpy as jnp
from jax.experimental import pallas as pl


def kernel(x, edge_index, W, b, gamma, beta):
    raise NotImplementedError("write your pallas kernel here")


# speedup vs baseline: 11.9001x; 11.9001x over previous
import functools

import jax
import jax.numpy as jnp
from jax import lax
from jax.experimental import pallas as pl
from jax.experimental.pallas import tpu as pltpu
from jax.experimental.pallas import tpu_sc as plsc

N = 10000
D = 128
E = 320000

NC = 2
NS = 16
NW = NC * NS

CH = 128
CPW = 80
EPW = CH * CPW
E_PAD = EPW * NW
ROWS_PER_TILE = 640
N_PAD = ROWS_PER_TILE * NS

_MESH = plsc.VectorSubcoreMesh(core_axis_name="c", subcore_axis_name="s",
                               num_cores=NC, num_subcores=NS)


def _deg_body(dstp, z1, degp0, degp1, deg_sh, ones_v, stage_v, didx, isem,
              ssem):
    cid = lax.axis_index("c")
    sid = lax.axis_index("s")
    wid = cid * NS + sid
    ebase = wid * EPW

    for j in range(CH // 16):
        ones_v[pl.ds(j * 16, 16)] = jnp.ones((16,), jnp.float32)
    row0 = sid * ROWS_PER_TILE
    pltpu.sync_copy(z1.at[pl.ds(0, ROWS_PER_TILE)], stage_v)
    pltpu.sync_copy(stage_v, deg_sh.at[pl.ds(row0, ROWS_PER_TILE)])
    plsc.subcore_barrier()

    nbuf = len(didx)
    for s in range(CPW // nbuf):
        descs = [
            pltpu.async_copy(dstp.at[pl.ds(ebase + (s * nbuf + k) * CH, CH)],
                             didx[k], isem) for k in range(nbuf)
        ]
        for d in descs:
            d.wait()
        descs = [
            pltpu.async_copy(ones_v, deg_sh.at[didx[k]], ssem, add=True)
            for k in range(nbuf)
        ]
        for d in descs:
            d.wait()

    plsc.subcore_barrier()
    pltpu.sync_copy(deg_sh.at[pl.ds(row0, ROWS_PER_TILE)], stage_v)

    @pl.when(cid == 0)
    def _():
        pltpu.sync_copy(stage_v, degp0.at[pl.ds(row0, ROWS_PER_TILE)])

    @pl.when(cid == 1)
    def _():
        pltpu.sync_copy(stage_v, degp1.at[pl.ds(row0, ROWS_PER_TILE)])


def _edge_body(srcp, dstp, y, z2, accp, sidx_all, didx0, didx1, rows0, rows1,
               acc_sh, isem0, isem1, gsem0, gsem1):
    cid = lax.axis_index("c")
    sid = lax.axis_index("s")
    wid = cid * NS + sid
    ebase = wid * EPW

    didx = (didx0, didx1)
    rows = (rows0, rows1)
    isem = (isem0, isem1)
    gsem = (gsem0, gsem1)
    QR = ROWS_PER_TILE // 5

    pltpu.sync_copy(srcp.at[pl.ds(ebase, EPW)], sidx_all)
    row0 = sid * ROWS_PER_TILE
    pltpu.sync_copy(z2.at[pl.ds(0, QR)], rows0)
    for k in range(5):
        pltpu.sync_copy(rows0, acc_sh.at[pl.ds(row0 + k * QR, QR)])
    plsc.subcore_barrier()

    def gather(c, b):
        pltpu.async_copy(y.at[sidx_all.at[pl.ds(c * CH, CH)]], rows[b],
                         gsem[b])

    def gather_wait(b):
        pltpu.make_async_copy(y.at[sidx_all.at[pl.ds(0, CH)]], rows[b],
                              gsem[b]).wait()

    def idx_fetch(c, b):
        pltpu.async_copy(dstp.at[pl.ds(ebase + c * CH, CH)], didx[b], isem[b])

    def idx_wait(b):
        pltpu.make_async_copy(dstp.at[pl.ds(ebase, CH)], didx[b],
                              isem[b]).wait()

    idx_fetch(0, 0)
    idx_fetch(1, 1)
    gather(0, 0)

    last = CPW - 1

    def step(g, _):
        for b in range(2):
            i = 2 * g + b
            gather(jnp.minimum(i + 1, last), 1 - b)
            gather_wait(b)
            idx_wait(b)
            pltpu.sync_copy(rows[b], acc_sh.at[didx[b]], add=True)
            idx_fetch(jnp.minimum(i + 2, last), b)
        return 0

    lax.fori_loop(0, CPW // 2, step, 0)

    gather_wait(0)
    idx_wait(0)
    idx_wait(1)

    plsc.subcore_barrier()
    for k in range(5):
        pltpu.sync_copy(acc_sh.at[pl.ds(row0 + k * QR, QR)], rows0)
        pltpu.sync_copy(rows0, accp.at[cid, pl.ds(row0 + k * QR, QR)])


def _scale_body(x_ref, w_ref, d0_ref, d1_ref, y_ref):
    deg = d0_ref[...] + d1_ref[...] + 1.0
    dinv = lax.rsqrt(deg)
    xw = jnp.dot(x_ref[...], w_ref[...], preferred_element_type=jnp.float32)
    y_ref[...] = xw * dinv


def _combine_body(a0_ref, a1_ref, y_ref, d0_ref, d1_ref, b_ref, out_ref,
                  st_ref):
    j = pl.program_id(0)
    deg = d0_ref[...] + d1_ref[...] + 1.0
    dinv = lax.rsqrt(deg)
    acc = a0_ref[0] + a1_ref[0] + y_ref[...]
    out = dinv * acc + b_ref[...]
    out_ref[...] = out
    st = jnp.stack([jnp.sum(out, axis=0), jnp.sum(out * out, axis=0)])

    @pl.when(j == 0)
    def _():
        st_ref[...] = st

    @pl.when(j > 0)
    def _():
        st_ref[...] += st


def _bn_body(out_ref, x_ref, st_ref, g_ref, bt_ref, f_ref):
    inv_n = 1.0 / N
    mean = st_ref[0:1, :] * inv_n
    var = st_ref[1:2, :] * inv_n - mean * mean
    rstd = lax.rsqrt(var + 1e-5)
    h = (out_ref[...] - mean) * rstd * g_ref[...] + bt_ref[...]
    h = jnp.where(h > 0, h, jnp.exp(h) - 1.0)
    f_ref[...] = h + x_ref[...]


def kernel(x, edge_index, W, b, gamma, beta):
    src = edge_index[0].astype(jnp.int32)
    dst = edge_index[1].astype(jnp.int32)
    npad = E_PAD - E
    srcp = jnp.concatenate([src, jnp.zeros((npad,), jnp.int32)])
    dstp = jnp.concatenate([dst, jnp.full((npad,), N, jnp.int32)])
    z1 = jnp.zeros((N_PAD,), jnp.float32)
    z2 = jnp.zeros((N_PAD, D), jnp.float32)

    deg_kernel = pl.kernel(
        _deg_body,
        out_type=[
            jax.ShapeDtypeStruct((N_PAD,), jnp.float32),
            jax.ShapeDtypeStruct((N_PAD,), jnp.float32),
        ],
        mesh=_MESH,
        scratch_types=[
            pltpu.VMEM_SHARED((N_PAD,), jnp.float32),
            pltpu.VMEM((CH,), jnp.float32),
            pltpu.VMEM((ROWS_PER_TILE,), jnp.float32),
            [pltpu.VMEM((CH,), jnp.int32) for _ in range(16)],
            pltpu.SemaphoreType.DMA,
            pltpu.SemaphoreType.DMA,
        ],
    )
    degp0, degp1 = deg_kernel(dstp, z1)
    degp0 = degp0.reshape(N_PAD, 1)
    degp1 = degp1.reshape(N_PAD, 1)

    R = 1000
    G = N // R
    dspec = pl.BlockSpec((R, 1), lambda j: (j, 0))
    y = pl.pallas_call(
        _scale_body,
        grid=(G,),
        in_specs=[
            pl.BlockSpec((R, D), lambda j: (j, 0)),
            pl.BlockSpec((D, D), lambda j: (0, 0)),
            dspec,
            dspec,
        ],
        out_specs=pl.BlockSpec((R, D), lambda j: (j, 0)),
        out_shape=jax.ShapeDtypeStruct((N, D), jnp.float32),
    )(x, W, degp0, degp1)

    edge_kernel = pl.kernel(
        _edge_body,
        out_type=jax.ShapeDtypeStruct((NC, N_PAD, D), jnp.float32),
        mesh=_MESH,
        scratch_types=[
            pltpu.VMEM((EPW,), jnp.int32),
            pltpu.VMEM((CH,), jnp.int32),
            pltpu.VMEM((CH,), jnp.int32),
            pltpu.VMEM((CH, D), jnp.float32),
            pltpu.VMEM((CH, D), jnp.float32),
            pltpu.VMEM_SHARED((N_PAD, D), jnp.float32),
            pltpu.SemaphoreType.DMA,
            pltpu.SemaphoreType.DMA,
            pltpu.SemaphoreType.DMA,
            pltpu.SemaphoreType.DMA,
        ],
    )
    accp = edge_kernel(srcp, dstp, y, z2)

    out, stats = pl.pallas_call(
        _combine_body,
        grid=(G,),
        in_specs=[
            pl.BlockSpec((1, R, D), lambda j: (0, j, 0)),
            pl.BlockSpec((1, R, D), lambda j: (1, j, 0)),
            pl.BlockSpec((R, D), lambda j: (j, 0)),
            dspec,
            dspec,
            pl.BlockSpec((1, D), lambda j: (0, 0)),
        ],
        out_specs=[
            pl.BlockSpec((R, D), lambda j: (j, 0)),
            pl.BlockSpec((2, D), lambda j: (0, 0)),
        ],
        out_shape=[
            jax.ShapeDtypeStruct((N, D), jnp.float32),
            jax.ShapeDtypeStruct((2, D), jnp.float32),
        ],
    )(accp, accp, y, degp0, degp1, b.reshape(1, D))

    final = pl.pallas_call(
        _bn_body,
        grid=(G,),
        in_specs=[
            pl.BlockSpec((R, D), lambda j: (j, 0)),
            pl.BlockSpec((R, D), lambda j: (j, 0)),
            pl.BlockSpec((2, D), lambda j: (0, 0)),
            pl.BlockSpec((1, D), lambda j: (0, 0)),
            pl.BlockSpec((1, D), lambda j: (0, 0)),
        ],
        out_specs=pl.BlockSpec((R, D), lambda j: (j, 0)),
        out_shape=jax.ShapeDtypeStruct((N, D), jnp.float32),
    )(out, x, stats, gamma.reshape(1, D), beta.reshape(1, D))

    return final


# speedup vs baseline: 33.5256x; 2.8173x over previous
import functools

import jax
import jax.numpy as jnp
from jax import lax
from jax.experimental import pallas as pl
from jax.experimental.pallas import tpu as pltpu
from jax.experimental.pallas import tpu_sc as plsc

N = 10000
D = 128
E = 320000

NC = 2
NS = 16
NW = NC * NS

CH = 80
CPW = 125
EPW = CH * CPW
ROWS_PER_TILE = 640
N_PAD = ROWS_PER_TILE * NS

_MESH = plsc.VectorSubcoreMesh(core_axis_name="c", subcore_axis_name="s",
                               num_cores=NC, num_subcores=NS)


def _deg_body(dstp, z1, degp0, degp1, deg_sh, ones_v, stage_v, didx, isem,
              ssem):
    cid = lax.axis_index("c")
    sid = lax.axis_index("s")
    wid = cid * NS + sid
    ebase = wid * EPW

    for j in range(CH // 16):
        ones_v[pl.ds(j * 16, 16)] = jnp.ones((16,), jnp.float32)
    row0 = sid * ROWS_PER_TILE
    pltpu.sync_copy(z1.at[pl.ds(0, ROWS_PER_TILE)], stage_v)
    pltpu.sync_copy(stage_v, deg_sh.at[pl.ds(row0, ROWS_PER_TILE)])
    plsc.subcore_barrier()

    nbuf = len(didx)
    done = 0
    while done < CPW:
        cur = min(nbuf, CPW - done)
        descs = [
            pltpu.async_copy(dstp.at[pl.ds(ebase + (done + k) * CH, CH)],
                             didx[k], isem) for k in range(cur)
        ]
        for d in descs:
            d.wait()
        descs = [
            pltpu.async_copy(ones_v, deg_sh.at[didx[k]], ssem, add=True)
            for k in range(cur)
        ]
        for d in descs:
            d.wait()
        done += cur

    plsc.subcore_barrier()
    pltpu.sync_copy(deg_sh.at[pl.ds(row0, ROWS_PER_TILE)], stage_v)

    @pl.when(cid == 0)
    def _():
        pltpu.sync_copy(stage_v, degp0.at[pl.ds(row0, ROWS_PER_TILE)])

    @pl.when(cid == 1)
    def _():
        pltpu.sync_copy(stage_v, degp1.at[pl.ds(row0, ROWS_PER_TILE)])


def _edge_body(srcp, dstp, y, z2, accp, sidx_all, didx0, didx1, rows0, rows1,
               acc_sh, isem0, isem1, gsem0, gsem1):
    cid = lax.axis_index("c")
    sid = lax.axis_index("s")
    wid = cid * NS + sid
    ebase = wid * EPW

    didx = (didx0, didx1)
    rows = (rows0, rows1)
    isem = (isem0, isem1)
    gsem = (gsem0, gsem1)
    QR = ROWS_PER_TILE // 8

    pltpu.sync_copy(srcp.at[pl.ds(ebase, EPW)], sidx_all)
    row0 = sid * ROWS_PER_TILE
    pltpu.sync_copy(z2.at[pl.ds(0, QR)], rows0)
    for k in range(8):
        pltpu.sync_copy(rows0, acc_sh.at[pl.ds(row0 + k * QR, QR)])
    plsc.subcore_barrier()

    def gather(c, b):
        pltpu.async_copy(y.at[sidx_all.at[pl.ds(c * CH, CH)]], rows[b],
                         gsem[b])

    def gather_wait(b):
        pltpu.make_async_copy(y.at[sidx_all.at[pl.ds(0, CH)]], rows[b],
                              gsem[b]).wait()

    def idx_fetch(c, b):
        pltpu.async_copy(dstp.at[pl.ds(ebase + c * CH, CH)], didx[b], isem[b])

    def idx_wait(b):
        pltpu.make_async_copy(dstp.at[pl.ds(ebase, CH)], didx[b],
                              isem[b]).wait()

    idx_fetch(0, 0)
    idx_fetch(1, 1)
    gather(0, 0)

    last = CPW - 1

    def step(g, _):
        for b in range(2):
            i = 2 * g + b
            gather(jnp.minimum(i + 1, last), 1 - b)
            gather_wait(b)
            idx_wait(b)
            pltpu.sync_copy(rows[b], acc_sh.at[didx[b]], add=True)
            idx_fetch(jnp.minimum(i + 2, last), b)
        return 0

    lax.fori_loop(0, CPW // 2, step, 0)

    gather_wait(0)
    idx_wait(0)
    pltpu.sync_copy(rows[0], acc_sh.at[didx[0]], add=True)
    idx_wait(1)

    plsc.subcore_barrier()
    for k in range(8):
        pltpu.sync_copy(acc_sh.at[pl.ds(row0 + k * QR, QR)], rows0)
        pltpu.sync_copy(rows0, accp.at[cid, pl.ds(row0 + k * QR, QR)])


def _scale_body(x_ref, w_ref, d0_ref, d1_ref, y_ref):
    deg = d0_ref[...] + d1_ref[...] + 1.0
    dinv = lax.rsqrt(deg)
    xw = jnp.dot(x_ref[...], w_ref[...], preferred_element_type=jnp.float32)
    y_ref[...] = xw * dinv


def _combine_body(a0_ref, a1_ref, y_ref, d0_ref, d1_ref, b_ref, out_ref,
                  st_ref):
    j = pl.program_id(0)
    deg = d0_ref[...] + d1_ref[...] + 1.0
    dinv = lax.rsqrt(deg)
    acc = a0_ref[0] + a1_ref[0] + y_ref[...]
    out = dinv * acc + b_ref[...]
    out_ref[...] = out
    st = jnp.stack([jnp.sum(out, axis=0), jnp.sum(out * out, axis=0)])

    @pl.when(j == 0)
    def _():
        st_ref[...] = st

    @pl.when(j > 0)
    def _():
        st_ref[...] += st


def _bn_body(out_ref, x_ref, st_ref, g_ref, bt_ref, f_ref):
    inv_n = 1.0 / N
    mean = st_ref[0:1, :] * inv_n
    var = st_ref[1:2, :] * inv_n - mean * mean
    rstd = lax.rsqrt(var + 1e-5)
    h = (out_ref[...] - mean) * rstd * g_ref[...] + bt_ref[...]
    h = jnp.where(h > 0, h, jnp.exp(h) - 1.0)
    f_ref[...] = h + x_ref[...]


def kernel(x, edge_index, W, b, gamma, beta):
    srcp = edge_index[0].astype(jnp.int32)
    dstp = edge_index[1].astype(jnp.int32)
    z1 = jnp.zeros((N_PAD,), jnp.float32)
    z2 = jnp.zeros((N_PAD, D), jnp.float32)

    deg_kernel = pl.kernel(
        _deg_body,
        out_type=[
            jax.ShapeDtypeStruct((N_PAD,), jnp.float32),
            jax.ShapeDtypeStruct((N_PAD,), jnp.float32),
        ],
        mesh=_MESH,
        scratch_types=[
            pltpu.VMEM_SHARED((N_PAD,), jnp.float32),
            pltpu.VMEM((CH,), jnp.float32),
            pltpu.VMEM((ROWS_PER_TILE,), jnp.float32),
            [pltpu.VMEM((CH,), jnp.int32) for _ in range(16)],
            pltpu.SemaphoreType.DMA,
            pltpu.SemaphoreType.DMA,
        ],
    )
    degp0, degp1 = deg_kernel(dstp, z1)
    degp0 = degp0.reshape(N_PAD, 1)
    degp1 = degp1.reshape(N_PAD, 1)

    R = 1000
    G = N // R
    dspec = pl.BlockSpec((R, 1), lambda j: (j, 0))
    y = pl.pallas_call(
        _scale_body,
        grid=(G,),
        in_specs=[
            pl.BlockSpec((R, D), lambda j: (j, 0)),
            pl.BlockSpec((D, D), lambda j: (0, 0)),
            dspec,
            dspec,
        ],
        out_specs=pl.BlockSpec((R, D), lambda j: (j, 0)),
        out_shape=jax.ShapeDtypeStruct((N, D), jnp.float32),
    )(x, W, degp0, degp1)

    edge_kernel = pl.kernel(
        _edge_body,
        out_type=jax.ShapeDtypeStruct((NC, N_PAD, D), jnp.float32),
        mesh=_MESH,
        scratch_types=[
            pltpu.VMEM((EPW,), jnp.int32),
            pltpu.VMEM((CH,), jnp.int32),
            pltpu.VMEM((CH,), jnp.int32),
            pltpu.VMEM((CH, D), jnp.float32),
            pltpu.VMEM((CH, D), jnp.float32),
            pltpu.VMEM_SHARED((N_PAD, D), jnp.float32),
            pltpu.SemaphoreType.DMA,
            pltpu.SemaphoreType.DMA,
            pltpu.SemaphoreType.DMA,
            pltpu.SemaphoreType.DMA,
        ],
    )
    accp = edge_kernel(srcp, dstp, y, z2)

    out, stats = pl.pallas_call(
        _combine_body,
        grid=(G,),
        in_specs=[
            pl.BlockSpec((1, R, D), lambda j: (0, j, 0)),
            pl.BlockSpec((1, R, D), lambda j: (1, j, 0)),
            pl.BlockSpec((R, D), lambda j: (j, 0)),
            dspec,
            dspec,
            pl.BlockSpec((1, D), lambda j: (0, 0)),
        ],
        out_specs=[
            pl.BlockSpec((R, D), lambda j: (j, 0)),
            pl.BlockSpec((2, D), lambda j: (0, 0)),
        ],
        out_shape=[
            jax.ShapeDtypeStruct((N, D), jnp.float32),
            jax.ShapeDtypeStruct((2, D), jnp.float32),
        ],
    )(accp, accp, y, degp0, degp1, b.reshape(1, D))

    final = pl.pallas_call(
        _bn_body,
        grid=(G,),
        in_specs=[
            pl.BlockSpec((R, D), lambda j: (j, 0)),
            pl.BlockSpec((R, D), lambda j: (j, 0)),
            pl.BlockSpec((2, D), lambda j: (0, 0)),
            pl.BlockSpec((1, D), lambda j: (0, 0)),
            pl.BlockSpec((1, D), lambda j: (0, 0)),
        ],
        out_specs=pl.BlockSpec((R, D), lambda j: (j, 0)),
        out_shape=jax.ShapeDtypeStruct((N, D), jnp.float32),
    )(out, x, stats, gamma.reshape(1, D), beta.reshape(1, D))

    return final


# speedup vs baseline: 36.8304x; 1.0986x over previous
import functools

import jax
import jax.numpy as jnp
from jax import lax
from jax.experimental import pallas as pl
from jax.experimental.pallas import tpu as pltpu
from jax.experimental.pallas import tpu_sc as plsc

N = 10000
D = 128
E = 320000

NC = 2
NS = 16
NW = NC * NS

CH = 80
CPW = 125
EPW = CH * CPW
ROWS_PER_TILE = 640
N_PAD = ROWS_PER_TILE * NS

_MESH = plsc.VectorSubcoreMesh(core_axis_name="c", subcore_axis_name="s",
                               num_cores=NC, num_subcores=NS)


def _deg_body(dstp, z1, degp0, degp1, deg_sh, ones_v, stage_v, didx, isem,
              ssem):
    cid = lax.axis_index("c")
    sid = lax.axis_index("s")
    wid = cid * NS + sid
    ebase = wid * EPW

    for j in range(CH // 16):
        ones_v[pl.ds(j * 16, 16)] = jnp.ones((16,), jnp.float32)
    row0 = sid * ROWS_PER_TILE
    pltpu.sync_copy(z1.at[pl.ds(0, ROWS_PER_TILE)], stage_v)
    pltpu.sync_copy(stage_v, deg_sh.at[pl.ds(row0, ROWS_PER_TILE)])
    plsc.subcore_barrier()

    nbuf = len(didx)
    done = 0
    while done < CPW:
        cur = min(nbuf, CPW - done)
        descs = [
            pltpu.async_copy(dstp.at[pl.ds(ebase + (done + k) * CH, CH)],
                             didx[k], isem) for k in range(cur)
        ]
        for d in descs:
            d.wait()
        descs = [
            pltpu.async_copy(ones_v, deg_sh.at[didx[k]], ssem, add=True)
            for k in range(cur)
        ]
        for d in descs:
            d.wait()
        done += cur

    plsc.subcore_barrier()
    pltpu.sync_copy(deg_sh.at[pl.ds(row0, ROWS_PER_TILE)], stage_v)

    @pl.when(cid == 0)
    def _():
        pltpu.sync_copy(stage_v, degp0.at[pl.ds(row0, ROWS_PER_TILE)])

    @pl.when(cid == 1)
    def _():
        pltpu.sync_copy(stage_v, degp1.at[pl.ds(row0, ROWS_PER_TILE)])


def _edge_body(srcp, dstp, y, z2, accp, sidx_all, didx0, didx1, didx2, rows0,
               rows1, rows2, acc_sh, isem0, isem1, isem2, gsem0, gsem1, gsem2,
               ssem0, ssem1, ssem2):
    cid = lax.axis_index("c")
    sid = lax.axis_index("s")
    wid = cid * NS + sid
    ebase = wid * EPW

    didx = (didx0, didx1, didx2)
    rows = (rows0, rows1, rows2)
    isem = (isem0, isem1, isem2)
    gsem = (gsem0, gsem1, gsem2)
    ssem = (ssem0, ssem1, ssem2)
    QR = ROWS_PER_TILE // 8

    pltpu.sync_copy(srcp.at[pl.ds(ebase, EPW)], sidx_all)
    row0 = sid * ROWS_PER_TILE
    pltpu.sync_copy(z2.at[pl.ds(0, QR)], rows0)
    for k in range(8):
        pltpu.sync_copy(rows0, acc_sh.at[pl.ds(row0 + k * QR, QR)])
    plsc.subcore_barrier()

    def gather(c, b):
        pltpu.async_copy(y.at[sidx_all.at[pl.ds(c * CH, CH)]], rows[b],
                         gsem[b])

    def gather_wait(b):
        pltpu.make_async_copy(y.at[sidx_all.at[pl.ds(0, CH)]], rows[b],
                              gsem[b]).wait()

    def idx_fetch(c, b):
        pltpu.async_copy(dstp.at[pl.ds(ebase + c * CH, CH)], didx[b], isem[b])

    def idx_wait(b):
        pltpu.make_async_copy(dstp.at[pl.ds(ebase, CH)], didx[b],
                              isem[b]).wait()

    def scatter(b):
        pltpu.async_copy(rows[b], acc_sh.at[didx[b]], ssem[b], add=True)

    def scatter_wait(b):
        pltpu.make_async_copy(rows[b], acc_sh.at[didx[b]], ssem[b]).wait()

    last = CPW - 1

    idx_fetch(0, 0)
    gather(0, 0)
    for i in (0, 1):
        idx_fetch(i + 1, (i + 1) % 3)
        gather(i + 1, (i + 1) % 3)
        gather_wait(i % 3)
        idx_wait(i % 3)
        scatter(i % 3)

    def step(g, _):
        for b3 in range(3):
            i = 3 * g + 2 + b3
            bi = (2 + b3) % 3
            bn = (b3) % 3
            scatter_wait(bn)
            idx_fetch(jnp.minimum(i + 1, last), bn)
            gather(jnp.minimum(i + 1, last), bn)
            gather_wait(bi)
            idx_wait(bi)
            scatter(bi)
        return 0

    lax.fori_loop(0, (CPW - 2) // 3, step, 0)

    scatter_wait(0)
    scatter_wait(1)
    gather_wait(2)
    idx_wait(2)

    plsc.subcore_barrier()
    for k in range(8):
        pltpu.sync_copy(acc_sh.at[pl.ds(row0 + k * QR, QR)], rows0)
        pltpu.sync_copy(rows0, accp.at[cid, pl.ds(row0 + k * QR, QR)])


def _scale_body(x_ref, w_ref, d0_ref, d1_ref, y_ref):
    deg = d0_ref[...] + d1_ref[...] + 1.0
    dinv = lax.rsqrt(deg)
    xw = jnp.dot(x_ref[...], w_ref[...], preferred_element_type=jnp.float32)
    y_ref[...] = xw * dinv


def _combine_body(a0_ref, a1_ref, y_ref, d0_ref, d1_ref, b_ref, out_ref,
                  st_ref):
    j = pl.program_id(0)
    deg = d0_ref[...] + d1_ref[...] + 1.0
    dinv = lax.rsqrt(deg)
    acc = a0_ref[0] + a1_ref[0] + y_ref[...]
    out = dinv * acc + b_ref[...]
    out_ref[...] = out
    st = jnp.stack([jnp.sum(out, axis=0), jnp.sum(out * out, axis=0)])

    @pl.when(j == 0)
    def _():
        st_ref[...] = st

    @pl.when(j > 0)
    def _():
        st_ref[...] += st


def _bn_body(out_ref, x_ref, st_ref, g_ref, bt_ref, f_ref):
    inv_n = 1.0 / N
    mean = st_ref[0:1, :] * inv_n
    var = st_ref[1:2, :] * inv_n - mean * mean
    rstd = lax.rsqrt(var + 1e-5)
    h = (out_ref[...] - mean) * rstd * g_ref[...] + bt_ref[...]
    h = jnp.where(h > 0, h, jnp.exp(h) - 1.0)
    f_ref[...] = h + x_ref[...]


def kernel(x, edge_index, W, b, gamma, beta):
    srcp = edge_index[0].astype(jnp.int32)
    dstp = edge_index[1].astype(jnp.int32)
    z1 = jnp.zeros((N_PAD,), jnp.float32)
    z2 = jnp.zeros((N_PAD, D), jnp.float32)

    deg_kernel = pl.kernel(
        _deg_body,
        out_type=[
            jax.ShapeDtypeStruct((N_PAD,), jnp.float32),
            jax.ShapeDtypeStruct((N_PAD,), jnp.float32),
        ],
        mesh=_MESH,
        scratch_types=[
            pltpu.VMEM_SHARED((N_PAD,), jnp.float32),
            pltpu.VMEM((CH,), jnp.float32),
            pltpu.VMEM((ROWS_PER_TILE,), jnp.float32),
            [pltpu.VMEM((CH,), jnp.int32) for _ in range(16)],
            pltpu.SemaphoreType.DMA,
            pltpu.SemaphoreType.DMA,
        ],
    )
    degp0, degp1 = deg_kernel(dstp, z1)
    degp0 = degp0.reshape(N_PAD, 1)
    degp1 = degp1.reshape(N_PAD, 1)

    R = 1000
    G = N // R
    dspec = pl.BlockSpec((R, 1), lambda j: (j, 0))
    y = pl.pallas_call(
        _scale_body,
        grid=(G,),
        in_specs=[
            pl.BlockSpec((R, D), lambda j: (j, 0)),
            pl.BlockSpec((D, D), lambda j: (0, 0)),
            dspec,
            dspec,
        ],
        out_specs=pl.BlockSpec((R, D), lambda j: (j, 0)),
        out_shape=jax.ShapeDtypeStruct((N, D), jnp.float32),
    )(x, W, degp0, degp1)

    edge_kernel = pl.kernel(
        _edge_body,
        out_type=jax.ShapeDtypeStruct((NC, N_PAD, D), jnp.float32),
        mesh=_MESH,
        scratch_types=[
            pltpu.VMEM((EPW,), jnp.int32),
            pltpu.VMEM((CH,), jnp.int32),
            pltpu.VMEM((CH,), jnp.int32),
            pltpu.VMEM((CH,), jnp.int32),
            pltpu.VMEM((CH, D), jnp.float32),
            pltpu.VMEM((CH, D), jnp.float32),
            pltpu.VMEM((CH, D), jnp.float32),
            pltpu.VMEM_SHARED((N_PAD, D), jnp.float32),
        ] + [pltpu.SemaphoreType.DMA] * 9,
    )
    accp = edge_kernel(srcp, dstp, y, z2)

    out, stats = pl.pallas_call(
        _combine_body,
        grid=(G,),
        in_specs=[
            pl.BlockSpec((1, R, D), lambda j: (0, j, 0)),
            pl.BlockSpec((1, R, D), lambda j: (1, j, 0)),
            pl.BlockSpec((R, D), lambda j: (j, 0)),
            dspec,
            dspec,
            pl.BlockSpec((1, D), lambda j: (0, 0)),
        ],
        out_specs=[
            pl.BlockSpec((R, D), lambda j: (j, 0)),
            pl.BlockSpec((2, D), lambda j: (0, 0)),
        ],
        out_shape=[
            jax.ShapeDtypeStruct((N, D), jnp.float32),
            jax.ShapeDtypeStruct((2, D), jnp.float32),
        ],
    )(accp, accp, y, degp0, degp1, b.reshape(1, D))

    final = pl.pallas_call(
        _bn_body,
        grid=(G,),
        in_specs=[
            pl.BlockSpec((R, D), lambda j: (j, 0)),
            pl.BlockSpec((R, D), lambda j: (j, 0)),
            pl.BlockSpec((2, D), lambda j: (0, 0)),
            pl.BlockSpec((1, D), lambda j: (0, 0)),
            pl.BlockSpec((1, D), lambda j: (0, 0)),
        ],
        out_specs=pl.BlockSpec((R, D), lambda j: (j, 0)),
        out_shape=jax.ShapeDtypeStruct((N, D), jnp.float32),
    )(out, x, stats, gamma.reshape(1, D), beta.reshape(1, D))

    return final
